# baseline (speedup 1.0000x reference)
import functools

import jax
import jax.numpy as jnp
from jax import lax
from jax.experimental import pallas as pl
from jax.experimental.pallas import tpu as pltpu
from jax.experimental.pallas import tpu_sc as plsc

_P = 16
_L = 576
_KEEP = 144
_NFILL = 432
_G = 24
_N = 64
_C = 3


def _rank_body(noise_ref, ids_ref, mask_ref):
    a = noise_ref[0, 0, :]
    col = a[:, None]
    row = a[None, :]
    il = lax.broadcasted_iota(jnp.int32, (_L, _L), 0)
    im_ = lax.broadcasted_iota(jnp.int32, (_L, _L), 1)
    cond = (row < col) | ((row == col) & (im_ < il))
    rank = jnp.sum(cond.astype(jnp.int32), axis=1)
    ids_ref[0, 0, :] = rank
    mask_ref[0, 0, :] = (rank >= _KEEP).astype(jnp.float32)


def _compute_ranks(noise):
    noise3 = noise.reshape(_N, 1, _L)
    ids3, mask3 = pl.pallas_call(
        _rank_body,
        grid=(_N,),
        in_specs=[pl.BlockSpec((1, 1, _L), lambda i: (i, 0, 0))],
        out_specs=[
            pl.BlockSpec((1, 1, _L), lambda i: (i, 0, 0)),
            pl.BlockSpec((1, 1, _L), lambda i: (i, 0, 0)),
        ],
        out_shape=[
            jax.ShapeDtypeStruct((_N, 1, _L), jnp.int32),
            jax.ShapeDtypeStruct((_N, 1, _L), jnp.float32),
        ],
    )(noise3)
    return ids3.reshape(_N, _L), mask3.reshape(_N, _L)


_NBLOCKS = _N * _G
_PER_W = _NBLOCKS // 32


@functools.cache
def _build_sc_masker():
    return functools.partial(
        pl.kernel,
        mesh=plsc.VectorSubcoreMesh(core_axis_name="c", subcore_axis_name="s"),
        compiler_params=pltpu.CompilerParams(needs_layout_passes=False, use_tc_tiling_on_sc=False),
        out_type=jax.ShapeDtypeStruct((_N * _C * 384 * 384,), jnp.float32),
        scratch_types=[
            pltpu.VMEM((18432,), jnp.float32),
            pltpu.VMEM((18432,), jnp.float32),
            pltpu.VMEM((_G, 768), jnp.float32),
            pltpu.VMEM((_G, 768), jnp.float32),
            pltpu.VMEM((1280,), jnp.int32),
            pltpu.VMEM((_PER_W, _G), jnp.int32),
            pltpu.SemaphoreType.DMA,
            pltpu.SemaphoreType.DMA,
            pltpu.SemaphoreType.DMA,
            pltpu.SemaphoreType.DMA,
            pltpu.SemaphoreType.DMA,
            pltpu.SemaphoreType.DMA,
        ],
    )(_sc_body)


def _sc_body(im_hbm, fill_hbm, rank_hbm, out_hbm,
             imb0, imb1, fb0, fb1, rkall, idxall,
             ims0, ims1, fls0, fls1, ous0, ous1):
    wid = lax.axis_index("s") * 2 + lax.axis_index("c")
    lane = lax.iota(jnp.int32, 16)
    n0 = wid * 2

    pltpu.sync_copy(rank_hbm.at[pl.ds(n0 * 640, 1280)], rkall)
    for lb in range(_PER_W):
        sl = lb // _G
        off = sl * 640 + (lb % _G) * _G
        nb = (n0 + sl) * _NFILL
        r0_ = rkall[pl.ds(off, 16)]
        r1_ = rkall[pl.ds(off + 8, 16)]
        idxall[lb, pl.ds(0, 16)] = jnp.maximum(r0_ - _KEEP, 0) + nb
        idxall[lb, pl.ds(8, 16)] = jnp.maximum(r1_ - _KEEP, 0) + nb

    def im_copies(i, imb, sem):
        n = n0 + i // _G
        hh = i % _G
        return [pltpu.make_async_copy(
            im_hbm.at[pl.ds(((n * _C + c) * _G + hh) * 6144, 6144)],
            imb.at[pl.ds(c * 6144, 6144)], sem)
            for c in range(_C)]

    def out_copies(i, imb, sem):
        n = n0 + i // _G
        hh = i % _G
        return [pltpu.make_async_copy(
            imb.at[pl.ds(c * 6144, 6144)],
            out_hbm.at[pl.ds(((n * _C + c) * _G + hh) * 6144, 6144)], sem)
            for c in range(_C)]

    def fill_copy(i, fb, sem):
        return pltpu.make_async_copy(fill_hbm.at[idxall.at[i]], fb, sem)

    def issue_in(i, imb, fb, ims, fls):
        for cp in im_copies(i, imb, ims):
            cp.start()
        fill_copy(i, fb, fls).start()

    def wait_in(i, imb, fb, ims, fls):
        for cp in im_copies(i, imb, ims):
            cp.wait()
        fill_copy(i, fb, fls).wait()

    def compute(i, imb, fb):
        rkoff = (i // _G) * 640 + (i % _G) * _G

        def patch_body(j, carry2):
            rkv = plsc.load_gather(rkall, [jnp.full((16,), rkoff + j,
                                                    jnp.int32)])
            msk = rkv >= _KEEP
            rz = jnp.minimum(rkv, 0)
            jv = jnp.full((16,), j, jnp.int32)
            for c in range(_C):
                src_col = lane * 3 + c + rz
                dst_col = lane + (j * _P + c * 6144)
                for p in range(_P):
                    src = plsc.load_gather(fb, [jv, src_col])
                    plsc.store_scatter(imb, [dst_col], src, mask=msk)
                    if p < _P - 1:
                        src_col = src_col + 48
                        dst_col = dst_col + 384
            return carry2

        lax.fori_loop(0, _G, patch_body, 0)

    def block_body(i, carry):
        wait = issue_in(i, imb0, fb0, ims0, fls0) or wait_in(i, imb0, fb0, ims0, fls0)
        compute(i, imb0, fb0)
        for cp in out_copies(i, imb0, ous0):
            cp.start()
        for cp in out_copies(i, imb0, ous0):
            cp.wait()
        return carry

    lax.fori_loop(0, _PER_W, block_body, 0)


def kernel(im, noise, fill_noise):
    ids_restore, mask = _compute_ranks(noise)
    rank_pad = jnp.pad(ids_restore, ((0, 1), (0, 64))).reshape(-1)
    fill_flat = fill_noise.reshape(_N * _NFILL, 768)
    masked_img = _build_sc_masker()(im.reshape(-1), fill_flat, rank_pad)
    return masked_img.reshape(_N, _C, 384, 384), mask, ids_restore


# speedup vs baseline: 1.3776x; 1.3776x over previous
import functools

import jax
import jax.numpy as jnp
from jax import lax
from jax.experimental import pallas as pl
from jax.experimental.pallas import tpu as pltpu
from jax.experimental.pallas import tpu_sc as plsc

_P = 16
_L = 576
_KEEP = 144
_NFILL = 432
_G = 24
_N = 64
_C = 3


def _rank_body(noise_ref, ids_ref, mask_ref):
    a = noise_ref[0, 0, :]
    col = a[:, None]
    row = a[None, :]
    il = lax.broadcasted_iota(jnp.int32, (_L, _L), 0)
    im_ = lax.broadcasted_iota(jnp.int32, (_L, _L), 1)
    cond = (row < col) | ((row == col) & (im_ < il))
    rank = jnp.sum(cond.astype(jnp.int32), axis=1)
    ids_ref[0, 0, :] = rank
    mask_ref[0, 0, :] = (rank >= _KEEP).astype(jnp.float32)


def _compute_ranks(noise):
    noise3 = noise.reshape(_N, 1, _L)
    ids3, mask3 = pl.pallas_call(
        _rank_body,
        grid=(_N,),
        in_specs=[pl.BlockSpec((1, 1, _L), lambda i: (i, 0, 0))],
        out_specs=[
            pl.BlockSpec((1, 1, _L), lambda i: (i, 0, 0)),
            pl.BlockSpec((1, 1, _L), lambda i: (i, 0, 0)),
        ],
        out_shape=[
            jax.ShapeDtypeStruct((_N, 1, _L), jnp.int32),
            jax.ShapeDtypeStruct((_N, 1, _L), jnp.float32),
        ],
    )(noise3)
    return ids3.reshape(_N, _L), mask3.reshape(_N, _L)


_NBLOCKS = _N * _G
_PER_W = _NBLOCKS // 32


@functools.cache
def _build_sc_masker():
    return functools.partial(
        pl.kernel,
        mesh=plsc.VectorSubcoreMesh(core_axis_name="c", subcore_axis_name="s"),
        compiler_params=pltpu.CompilerParams(needs_layout_passes=False),
        out_type=jax.ShapeDtypeStruct((_N, _C, 384, 384), jnp.float32),
        scratch_types=[
            pltpu.VMEM((_C * _P, 384), jnp.float32),
            pltpu.VMEM((_G, 768), jnp.float32),
            pltpu.VMEM((1280,), jnp.int32),
            pltpu.VMEM((_PER_W, _G), jnp.int32),
            pltpu.SemaphoreType.DMA,
        ],
    )(_sc_body)


def _sc_body(im_hbm, fill_hbm, rank_hbm, out_hbm, imb, fb, rkall, idxall, sem):
    wid = lax.axis_index("s") * 2 + lax.axis_index("c")
    lane = lax.iota(jnp.int32, 16)
    n0 = wid * 2

    pltpu.sync_copy(rank_hbm.at[pl.ds(n0 * 640, 1280)], rkall)
    for lb in range(_PER_W):
        sl = lb // _G
        off = sl * 640 + (lb % _G) * _G
        nb = (n0 + sl) * _NFILL
        r0_ = rkall[pl.ds(off, 16)]
        r1_ = rkall[pl.ds(off + 8, 16)]
        idxall[lb, pl.ds(0, 16)] = jnp.maximum(r0_ - _KEEP, 0) + nb
        idxall[lb, pl.ds(8, 16)] = jnp.maximum(r1_ - _KEEP, 0) + nb

    def block_body(i, carry):
        n = n0 + i // _G
        hh = i % _G
        for c in range(_C):
            pltpu.sync_copy(im_hbm.at[n, c, pl.ds(hh * _P, _P), :],
                            imb.at[pl.ds(c * _P, _P)])
        pltpu.async_copy(fill_hbm.at[idxall.at[i]], fb, sem).wait()
        rkoff = (i // _G) * 640 + (i % _G) * _G

        def patch_body(j, carry2):
            rkv = plsc.load_gather(rkall, [jnp.full((16,), rkoff + j,
                                                    jnp.int32)])
            msk = rkv >= _KEEP
            rz = jnp.minimum(rkv, 0)
            jv = jnp.full((16,), j, jnp.int32)
            colidx = lane + j * _P
            for c in range(_C):
                src_col = lane * 3 + c + rz
                row_v = jnp.full((16,), c * _P, jnp.int32) + rz
                for p in range(_P):
                    src = plsc.load_gather(fb, [jv, src_col])
                    plsc.store_scatter(imb, [row_v, colidx], src, mask=msk)
                    if p < _P - 1:
                        src_col = src_col + 48
                        row_v = row_v + 1
            return carry2

        lax.fori_loop(0, _G, patch_body, 0)
        for c in range(_C):
            pltpu.sync_copy(imb.at[pl.ds(c * _P, _P)],
                            out_hbm.at[n, c, pl.ds(hh * _P, _P), :])
        return carry

    lax.fori_loop(0, _PER_W, block_body, 0)


def kernel(im, noise, fill_noise):
    ids_restore, mask = _compute_ranks(noise)
    rank_pad = jnp.pad(ids_restore, ((0, 1), (0, 64))).reshape(-1)
    fill_flat = fill_noise.reshape(_N * _NFILL, 768)
    masked_img = _build_sc_masker()(im, fill_flat, rank_pad)
    return masked_img, mask, ids_restore


# speedup vs baseline: 2.0099x; 1.4590x over previous
import functools

import jax
import jax.numpy as jnp
from jax import lax
from jax.experimental import pallas as pl
from jax.experimental.pallas import tpu as pltpu
from jax.experimental.pallas import tpu_sc as plsc

_P = 16
_L = 576
_KEEP = 144
_NFILL = 432
_G = 24
_N = 64
_C = 3


def _rank_body(noise_ref, ids_ref, mask_ref):
    a = noise_ref[0, 0, :]
    col = a[:, None]
    row = a[None, :]
    il = lax.broadcasted_iota(jnp.int32, (_L, _L), 0)
    im_ = lax.broadcasted_iota(jnp.int32, (_L, _L), 1)
    cond = (row < col) | ((row == col) & (im_ < il))
    rank = jnp.sum(cond.astype(jnp.int32), axis=1)
    ids_ref[0, 0, :] = rank
    mask_ref[0, 0, :] = (rank >= _KEEP).astype(jnp.float32)


def _compute_ranks(noise):
    noise3 = noise.reshape(_N, 1, _L)
    ids3, mask3 = pl.pallas_call(
        _rank_body,
        grid=(_N,),
        in_specs=[pl.BlockSpec((1, 1, _L), lambda i: (i, 0, 0))],
        out_specs=[
            pl.BlockSpec((1, 1, _L), lambda i: (i, 0, 0)),
            pl.BlockSpec((1, 1, _L), lambda i: (i, 0, 0)),
        ],
        out_shape=[
            jax.ShapeDtypeStruct((_N, 1, _L), jnp.int32),
            jax.ShapeDtypeStruct((_N, 1, _L), jnp.float32),
        ],
    )(noise3)
    return ids3.reshape(_N, _L), mask3.reshape(_N, _L)


_NBLOCKS = _N * _G
_PER_W = _NBLOCKS // 32


@functools.cache
def _build_sc_masker():
    return functools.partial(
        pl.kernel,
        mesh=plsc.VectorSubcoreMesh(core_axis_name="c", subcore_axis_name="s"),
        compiler_params=pltpu.CompilerParams(needs_layout_passes=False),
        out_type=jax.ShapeDtypeStruct((_N, _C, 384, 384), jnp.float32),
        scratch_types=[
            pltpu.VMEM((_C * _P, 384), jnp.float32),
            pltpu.VMEM((_C * _P, 384), jnp.float32),
            pltpu.VMEM((_G, 768), jnp.float32),
            pltpu.VMEM((_G, 768), jnp.float32),
            pltpu.VMEM((1280,), jnp.int32),
            pltpu.VMEM((_PER_W, _G), jnp.int32),
            pltpu.SemaphoreType.DMA,
            pltpu.SemaphoreType.DMA,
            pltpu.SemaphoreType.DMA,
            pltpu.SemaphoreType.DMA,
            pltpu.SemaphoreType.DMA,
            pltpu.SemaphoreType.DMA,
        ],
    )(_sc_body)


def _sc_body(im_hbm, fill_hbm, rank_hbm, out_hbm,
             imb0, imb1, fb0, fb1, rkall, idxall,
             ims0, ims1, fls0, fls1, ous0, ous1):
    wid = lax.axis_index("s") * 2 + lax.axis_index("c")
    lane = lax.iota(jnp.int32, 16)
    n0 = wid * 2

    pltpu.sync_copy(rank_hbm.at[pl.ds(n0 * 640, 1280)], rkall)
    for lb in range(_PER_W):
        sl = lb // _G
        off = sl * 640 + (lb % _G) * _G
        nb = (n0 + sl) * _NFILL
        r0_ = rkall[pl.ds(off, 16)]
        r1_ = rkall[pl.ds(off + 8, 16)]
        idxall[lb, pl.ds(0, 16)] = jnp.maximum(r0_ - _KEEP, 0) + nb
        idxall[lb, pl.ds(8, 16)] = jnp.maximum(r1_ - _KEEP, 0) + nb

    def im_copies(i, imb, sem):
        n = n0 + i // _G
        hh = i % _G
        return [pltpu.make_async_copy(
            im_hbm.at[n, c, pl.ds(hh * _P, _P), :],
            imb.at[pl.ds(c * _P, _P)], sem) for c in range(_C)]

    def out_copies(i, imb, sem):
        n = n0 + i // _G
        hh = i % _G
        return [pltpu.make_async_copy(
            imb.at[pl.ds(c * _P, _P)],
            out_hbm.at[n, c, pl.ds(hh * _P, _P), :], sem) for c in range(_C)]

    def issue_in(i, imb, fb, ims, fls):
        for cp in im_copies(i, imb, ims):
            cp.start()
        pltpu.make_async_copy(fill_hbm.at[idxall.at[i]], fb, fls).start()

    def wait_in(i, imb, fb, ims, fls):
        for cp in im_copies(i, imb, ims):
            cp.wait()
        pltpu.make_async_copy(fill_hbm.at[idxall.at[i]], fb, fls).wait()

    def compute(i, imb, fb):
        rkoff = (i // _G) * 640 + (i % _G) * _G

        def patch_body(j, carry2):
            rkv = plsc.load_gather(rkall, [jnp.full((16,), rkoff + j,
                                                    jnp.int32)])
            msk = rkv >= _KEEP
            rz = jnp.minimum(rkv, 0)
            jv = jnp.full((16,), j, jnp.int32)
            colidx = lane + j * _P
            for c in range(_C):
                src_col = lane * 3 + c + rz
                row_v = jnp.full((16,), c * _P, jnp.int32) + rz
                for p in range(_P):
                    src = plsc.load_gather(fb, [jv, src_col])
                    plsc.store_scatter(imb, [row_v, colidx], src, mask=msk)
                    if p < _P - 1:
                        src_col = src_col + 48
                        row_v = row_v + 1
            return carry2

        lax.fori_loop(0, _G, patch_body, 0)

    issue_in(0, imb0, fb0, ims0, fls0)
    issue_in(1, imb1, fb1, ims1, fls1)

    def pipe_body(k, carry):
        i0 = 2 * k
        i1 = 2 * k + 1
        wait_in(i0, imb0, fb0, ims0, fls0)
        compute(i0, imb0, fb0)
        for cp in out_copies(i0, imb0, ous0):
            cp.start()
        wait_in(i1, imb1, fb1, ims1, fls1)
        compute(i1, imb1, fb1)
        for cp in out_copies(i1, imb1, ous1):
            cp.start()

        @pl.when(k < _PER_W // 2 - 1)
        def _prefetch():
            for cp in out_copies(i0, imb0, ous0):
                cp.wait()
            issue_in(i0 + 2, imb0, fb0, ims0, fls0)
            for cp in out_copies(i1, imb1, ous1):
                cp.wait()
            issue_in(i1 + 2, imb1, fb1, ims1, fls1)

        return carry

    lax.fori_loop(0, _PER_W // 2, pipe_body, 0)
    for cp in out_copies(_PER_W - 2, imb0, ous0):
        cp.wait()
    for cp in out_copies(_PER_W - 1, imb1, ous1):
        cp.wait()


def kernel(im, noise, fill_noise):
    ids_restore, mask = _compute_ranks(noise)
    rank_pad = jnp.pad(ids_restore, ((0, 1), (0, 64))).reshape(-1)
    fill_flat = fill_noise.reshape(_N * _NFILL, 768)
    masked_img = _build_sc_masker()(im, fill_flat, rank_pad)
    return masked_img, mask, ids_restore
